# Initial kernel scaffold; baseline (speedup 1.0000x reference)
import jax
import jax.numpy as jnp
from jax.experimental import pallas as pl


def kernel(x, edge_index3, edge_index2, edge_attr, batch, batch_edge, node_batch_edge, num_node, params):
    raise NotImplementedError("write your pallas kernel here")


# speedup vs baseline: 8.7328x; 8.7328x over previous
import functools

import jax
import jax.numpy as jnp
from jax import lax
from jax.experimental import pallas as pl
from jax.experimental.pallas import tpu as pltpu
from jax.experimental.pallas import tpu_sc as plsc

_N = 10000
_E = 160000
_T = 1280000
_B = 16
_NC = 2
_NS = 16
_NW = _NC * _NS

_BLK_E = 4000
_BLK_N = 2000

_f32 = jnp.float32


def _mesh():
    return plsc.VectorSubcoreMesh(core_axis_name="c", subcore_axis_name="s")


_SC_PARAMS = pltpu.CompilerParams(use_tc_tiling_on_sc=False)


def _edge_lin_body(c_ref, w1_ref, b1_ref, w2_ref, b2_ref, c1_ref, c2_ref, had_ref):
    c = c_ref[...]
    c1 = jnp.dot(c, w1_ref[...], preferred_element_type=_f32) + b1_ref[...]
    c2 = jnp.dot(c, w2_ref[...], preferred_element_type=_f32) + b2_ref[...]
    c1_ref[...] = c1
    c2_ref[...] = c2
    had_ref[...] = c1 * c2


def _edge_lin(C, W1, b1, W2, b2):
    d = C.shape[1]
    grid = (_E // _BLK_E,)
    bs_c = pl.BlockSpec((_BLK_E, d), lambda i: (i, 0))
    bs_w = pl.BlockSpec((d, d), lambda i: (0, 0))
    bs_b = pl.BlockSpec((1, d), lambda i: (0, 0))
    return pl.pallas_call(
        _edge_lin_body,
        grid=grid,
        in_specs=[bs_c, bs_w, bs_b, bs_w, bs_b],
        out_specs=[bs_c, bs_c, bs_c],
        out_shape=[jax.ShapeDtypeStruct((_E, d), _f32)] * 3,
    )(C, W1, b1.reshape(1, d), W2, b2.reshape(1, d))


def _node_lin_body(x_ref, w3_ref, b3_ref, w4_ref, b4_ref, xw3_ref, xm_ref):
    xv = x_ref[...]
    xw3_ref[...] = jnp.dot(xv, w3_ref[...], preferred_element_type=_f32) + b3_ref[...]
    xm_ref[...] = jnp.dot(xv, w4_ref[...], preferred_element_type=_f32) + b4_ref[...]


def _node_lin(x, W3, b3, W4, b4):
    dn, d = W3.shape
    grid = (_N // _BLK_N,)
    bs_x = pl.BlockSpec((_BLK_N, dn), lambda i: (i, 0))
    bs_o = pl.BlockSpec((_BLK_N, d), lambda i: (i, 0))
    bs_w = pl.BlockSpec((dn, d), lambda i: (0, 0))
    bs_b = pl.BlockSpec((1, d), lambda i: (0, 0))
    return pl.pallas_call(
        _node_lin_body,
        grid=grid,
        in_specs=[bs_x, bs_w, bs_b, bs_w, bs_b],
        out_specs=[bs_o, bs_o],
        out_shape=[jax.ShapeDtypeStruct((_N, d), _f32)] * 2,
    )(x, W3, b3.reshape(1, d), W4, b4.reshape(1, d))


def _edge_out_body(c_ref, mm_ref, had_ref, dg_ref, id_ref, wa, wb, wc, wd, be,
                   out_ref, *, apply_gelu):
    acc = jnp.dot(c_ref[...], wa[...], preferred_element_type=_f32)
    acc += jnp.dot(mm_ref[...], wb[...], preferred_element_type=_f32)
    acc += jnp.dot(had_ref[...], wc[...], preferred_element_type=_f32)
    acc += jnp.dot(dg_ref[...] * id_ref[...], wd[...], preferred_element_type=_f32)
    acc += be[...]
    out_ref[...] = jax.nn.gelu(acc) if apply_gelu else acc


def _edge_out(C, mm, had, dg_raw, ident, We, be, apply_gelu):
    d = C.shape[1]
    do = We.shape[1]
    grid = (_E // _BLK_E,)
    bs_in = pl.BlockSpec((_BLK_E, d), lambda i: (i, 0))
    bs_id = pl.BlockSpec((_BLK_E, 1), lambda i: (i, 0))
    bs_w = pl.BlockSpec((d, do), lambda i: (0, 0))
    bs_b = pl.BlockSpec((1, do), lambda i: (0, 0))
    bs_o = pl.BlockSpec((_BLK_E, do), lambda i: (i, 0))
    return pl.pallas_call(
        functools.partial(_edge_out_body, apply_gelu=apply_gelu),
        grid=grid,
        in_specs=[bs_in, bs_in, bs_in, bs_in, bs_id, bs_w, bs_w, bs_w, bs_w, bs_b],
        out_specs=bs_o,
        out_shape=jax.ShapeDtypeStruct((_E, do), _f32),
    )(C, mm, had, dg_raw, ident, We[:d], We[d:2 * d], We[2 * d:3 * d], We[3 * d:], be.reshape(1, do))


def _node_out_body(x_ref, ag_ref, wa, wb, bn, out_ref, *, apply_gelu):
    acc = jnp.dot(x_ref[...], wa[...], preferred_element_type=_f32)
    agg = ag_ref[0] + ag_ref[1]
    acc += jnp.dot(agg, wb[...], preferred_element_type=_f32)
    acc += bn[...]
    out_ref[...] = jax.nn.gelu(acc) if apply_gelu else acc


def _node_out(x, agg2, Wn, bn, apply_gelu):
    dn = x.shape[1]
    d = agg2.shape[2]
    do = Wn.shape[1]
    grid = (_N // _BLK_N,)
    bs_x = pl.BlockSpec((_BLK_N, dn), lambda i: (i, 0))
    bs_a = pl.BlockSpec((2, _BLK_N, d), lambda i: (0, i, 0))
    bs_o = pl.BlockSpec((_BLK_N, do), lambda i: (i, 0))
    return pl.pallas_call(
        functools.partial(_node_out_body, apply_gelu=apply_gelu),
        grid=grid,
        in_specs=[bs_x,
                  bs_a,
                  pl.BlockSpec((dn, do), lambda i: (0, 0)),
                  pl.BlockSpec((d, do), lambda i: (0, 0)),
                  pl.BlockSpec((1, do), lambda i: (0, 0))],
        out_specs=bs_o,
        out_shape=jax.ShapeDtypeStruct((_N, do), _f32),
    )(x, agg2, Wn[:dn], Wn[dn:], bn.reshape(1, do))


def _seg_nodes_body(b_ref, x_ref, out_ref):
    i = pl.program_id(0)
    bid = lax.broadcasted_iota(jnp.int32, (1, _B), 1).astype(_f32)
    ind = (b_ref[...] == bid).astype(_f32)
    part = lax.dot_general(ind, x_ref[...], (((0,), (0,)), ((), ())),
                           preferred_element_type=_f32)

    @pl.when(i == 0)
    def _():
        out_ref[...] = jnp.zeros_like(out_ref)

    out_ref[...] += part


def _seg_nodes(batch_col, ox):
    dx = ox.shape[1]
    grid = (_N // _BLK_N,)
    return pl.pallas_call(
        _seg_nodes_body,
        grid=grid,
        in_specs=[pl.BlockSpec((_BLK_N, 1), lambda i: (i, 0)),
                  pl.BlockSpec((_BLK_N, dx), lambda i: (i, 0))],
        out_specs=pl.BlockSpec((_B, dx), lambda i: (0, 0)),
        out_shape=jax.ShapeDtypeStruct((_B, dx), _f32),
    )(batch_col, ox)


def _seg_edges_body(b_ref, id_ref, c_ref, hd_ref, ho_ref):
    i = pl.program_id(0)
    bid = lax.broadcasted_iota(jnp.int32, (1, _B), 1).astype(_f32)
    ind = (b_ref[...] == bid).astype(_f32)
    cv = c_ref[...]
    wc = cv * id_ref[...]
    s1 = lax.dot_general(ind, cv, (((0,), (0,)), ((), ())),
                         preferred_element_type=_f32)
    s2 = lax.dot_general(ind, wc, (((0,), (0,)), ((), ())),
                         preferred_element_type=_f32)

    @pl.when(i == 0)
    def _():
        hd_ref[...] = jnp.zeros_like(hd_ref)
        ho_ref[...] = jnp.zeros_like(ho_ref)

    hd_ref[...] += s2
    ho_ref[...] += s1 - s2


def _seg_edges(batch_col, ident, oc):
    dc = oc.shape[1]
    grid = (_E // _BLK_E,)
    return pl.pallas_call(
        _seg_edges_body,
        grid=grid,
        in_specs=[pl.BlockSpec((_BLK_E, 1), lambda i: (i, 0)),
                  pl.BlockSpec((_BLK_E, 1), lambda i: (i, 0)),
                  pl.BlockSpec((_BLK_E, dc), lambda i: (i, 0))],
        out_specs=[pl.BlockSpec((_B, dc), lambda i: (0, 0)),
                   pl.BlockSpec((_B, dc), lambda i: (0, 0))],
        out_shape=[jax.ShapeDtypeStruct((_B, dc), _f32)] * 2,
    )(batch_col, ident, oc)


def _mlp_body(hx_ref, hd_ref, ho_ref, w0a, w0b, w0c, b0, w1, b1, w2, b2, out_ref):
    h = jnp.dot(hx_ref[...], w0a[...], preferred_element_type=_f32)
    h += jnp.dot(hd_ref[...], w0b[...], preferred_element_type=_f32)
    h += jnp.dot(ho_ref[...], w0c[...], preferred_element_type=_f32)
    h = jax.nn.gelu(h + b0[...])
    h = jax.nn.gelu(jnp.dot(h, w1[...], preferred_element_type=_f32) + b1[...])
    out_ref[...] = jnp.dot(h, w2[...], preferred_element_type=_f32) + b2[...]


def _mlp(hx, hd, ho, params):
    dx = hx.shape[1]
    dc = hd.shape[1]
    w0 = params['fc0_W']
    outs = jax.ShapeDtypeStruct((_B, 1), _f32)
    return pl.pallas_call(
        _mlp_body,
        out_shape=outs,
    )(hx, hd, ho, w0[:dx], w0[dx:dx + dc], w0[dx + dc:],
      params['fc0_b'].reshape(1, -1),
      params['fc1_W'], params['fc1_b'].reshape(1, -1),
      params['fc2_W'], params['fc2_b'].reshape(1, -1))


_CH = 128
_T_CHUNKS = _T // _CH
_E_CHUNKS = _E // _CH


def _make_sc_prod(d):
    nch = _T_CHUNKS // _NW

    @functools.partial(
        pl.kernel,
        mesh=_mesh(),
        compiler_params=_SC_PARAMS,
        out_type=jax.ShapeDtypeStruct((_T, d), _f32),
        scratch_types=[pltpu.VMEM((_CH,), jnp.int32),
                       pltpu.VMEM((_CH,), jnp.int32),
                       pltpu.VMEM((_CH, d), _f32),
                       pltpu.VMEM((_CH, d), _f32)],
    )
    def k(ti_hbm, tj_hbm, c1_hbm, c2_hbm, prod_hbm, ib, jb, c1b, c2b):
        wid = lax.axis_index("s") * _NC + lax.axis_index("c")
        n_mine = _T_CHUNKS // _NW + jnp.where(wid < (_T_CHUNKS % _NW), 1, 0)

        @pl.loop(0, n_mine)
        def _(ci):
            off = (ci * _NW + wid) * _CH
            pltpu.sync_copy(ti_hbm.at[pl.ds(off, _CH)], ib)
            pltpu.sync_copy(tj_hbm.at[pl.ds(off, _CH)], jb)
            pltpu.sync_copy(c1_hbm.at[ib], c1b)
            pltpu.sync_copy(c2_hbm.at[jb], c2b)

            @pl.loop(0, _CH)
            def _(r):
                for q in range(d // 16):
                    sl = pl.ds(q * 16, 16)
                    c1b[r, sl] = c1b[r, sl] * c2b[r, sl]

            pltpu.sync_copy(c1b, prod_hbm.at[pl.ds(off, _CH), :])

    return k


def _make_sc_scatter_mm(d):
    nb = d // 8
    nbc = nb // _NC
    per_tile = _T_CHUNKS // _NS

    @functools.partial(
        pl.kernel,
        mesh=_mesh(),
        compiler_params=_SC_PARAMS,
        out_type=jax.ShapeDtypeStruct((_E, d), _f32),
        scratch_types=[pltpu.VMEM((_CH,), jnp.int32),
                       pltpu.VMEM((_CH, 8), _f32),
                       pltpu.VMEM_SHARED((_E, 8), _f32)],
    )
    def k(tk_hbm, prod_hbm, zero_hbm, mm_hbm, kb, pb, acc):
        core = lax.axis_index("c")
        sub = lax.axis_index("s")
        rows = _E // _NS
        for g in range(nbc):
            bk = core * nbc + g

            @pl.when(sub == 0)
            def _():
                pltpu.sync_copy(zero_hbm, acc)

            plsc.subcore_barrier()

            @pl.loop(0, per_tile)
            def _(ci):
                off = (ci * _NS + sub) * _CH
                pltpu.sync_copy(tk_hbm.at[pl.ds(off, _CH)], kb)
                pltpu.sync_copy(prod_hbm.at[pl.ds(off, _CH), pl.ds(bk * 8, 8)], pb)
                pltpu.sync_copy(pb, acc.at[kb], add=True)

            plsc.subcore_barrier()
            pltpu.sync_copy(acc.at[pl.ds(sub * rows, rows)],
                            mm_hbm.at[pl.ds(sub * rows, rows), pl.ds(bk * 8, 8)])
            plsc.subcore_barrier()

    return k


def _make_sc_edge(d):
    nrow = _N // _NS

    @functools.partial(
        pl.kernel,
        mesh=_mesh(),
        compiler_params=_SC_PARAMS,
        out_type=[jax.ShapeDtypeStruct((_E, d), _f32),
                  jax.ShapeDtypeStruct((_NC, _N, d), _f32)],
        scratch_types=[pltpu.VMEM((_CH,), jnp.int32),
                       pltpu.VMEM((_CH,), jnp.int32),
                       pltpu.VMEM((_CH, d), _f32),
                       pltpu.VMEM((_CH, d), _f32),
                       pltpu.VMEM_SHARED((_N, d), _f32)],
    )
    def k(src_hbm, dst_hbm, xw3_hbm, xm_hbm, c_hbm, zero_hbm, dg_hbm, agg_hbm,
          sb, db, gb, cb, acc):
        core = lax.axis_index("c")
        sub = lax.axis_index("s")
        wid = sub * _NC + core
        n_mine = _E_CHUNKS // _NW + jnp.where(wid < (_E_CHUNKS % _NW), 1, 0)

        @pl.when(sub == 0)
        def _():
            pltpu.sync_copy(zero_hbm, acc)

        plsc.subcore_barrier()

        @pl.loop(0, n_mine)
        def _(ci):
            off = (ci * _NW + wid) * _CH
            pltpu.sync_copy(src_hbm.at[pl.ds(off, _CH)], sb)
            pltpu.sync_copy(dst_hbm.at[pl.ds(off, _CH)], db)
            pltpu.sync_copy(xw3_hbm.at[sb], gb)
            pltpu.sync_copy(gb, dg_hbm.at[pl.ds(off, _CH), :])
            pltpu.sync_copy(xm_hbm.at[db], gb)
            pltpu.sync_copy(c_hbm.at[pl.ds(off, _CH), :], cb)

            @pl.loop(0, _CH)
            def _(r):
                for q in range(d // 16):
                    sl = pl.ds(q * 16, 16)
                    cb[r, sl] = cb[r, sl] * gb[r, sl]

            pltpu.sync_copy(cb, acc.at[sb], add=True)

        plsc.subcore_barrier()
        pltpu.sync_copy(acc.at[pl.ds(sub * nrow, nrow)],
                        agg_hbm.at[core, pl.ds(sub * nrow, nrow), :])

    return k


def kernel(x, edge_index3, edge_index2, edge_attr, batch, batch_edge,
           node_batch_edge, num_node, params):
    del node_batch_edge, num_node
    ti = edge_index3[0].astype(jnp.int32)
    tj = edge_index3[1].astype(jnp.int32)
    tk = edge_index3[2].astype(jnp.int32)
    src = edge_index2[0].astype(jnp.int32)
    dst = edge_index2[1].astype(jnp.int32)
    ident = edge_attr[:, 0:1]
    zeros_e8 = jnp.zeros((_E, 8), _f32)

    sc_prod = {16: _make_sc_prod(16), 64: _make_sc_prod(64)}
    sc_mm = {16: _make_sc_scatter_mm(16), 64: _make_sc_scatter_mm(64)}
    sc_edge = {16: _make_sc_edge(16), 64: _make_sc_edge(64)}

    C = edge_attr
    xc = x
    out_x = [x]
    out_C = [C]
    for i in range(5):
        d = C.shape[1]
        apply_gelu = i < 4
        c1, c2, had = _edge_lin(C, params['l%d_W1' % i], params['l%d_b1' % i],
                                params['l%d_W2' % i], params['l%d_b2' % i])
        xw3, xm = _node_lin(xc, params['l%d_W3' % i], params['l%d_b3' % i],
                            params['l%d_W4' % i], params['l%d_b4' % i])
        prod = sc_prod[d](ti, tj, c1, c2)
        mm = sc_mm[d](tk, prod, zeros_e8)
        dg_raw, agg2 = sc_edge[d](src, dst, xw3, xm, C, jnp.zeros((_N, d), _f32))
        C = _edge_out(C, mm, had, dg_raw, ident,
                      params['l%d_We' % i], params['l%d_be' % i], apply_gelu)
        xc = _node_out(xc, agg2, params['l%d_Wn' % i], params['l%d_bn' % i],
                       apply_gelu)
        out_x.append(xc)
        out_C.append(C)

    ox = jnp.concatenate(out_x, axis=1)
    oc = jnp.concatenate(out_C, axis=1)
    batch_col = batch.astype(_f32).reshape(_N, 1)
    batch_e_col = batch_edge.astype(_f32).reshape(_E, 1)
    hx = _seg_nodes(batch_col, ox)
    hd, ho = _seg_edges(batch_e_col, ident, oc)
    return _mlp(hx, hd, ho, params)


# speedup vs baseline: 16.2921x; 1.8656x over previous
import functools

import jax
import jax.numpy as jnp
from jax import lax
from jax.experimental import pallas as pl
from jax.experimental.pallas import tpu as pltpu
from jax.experimental.pallas import tpu_sc as plsc

_N = 10000
_E = 160000
_T = 1280000
_B = 16
_NC = 2
_NS = 16
_NW = _NC * _NS

_BLK_E = 4000
_BLK_N = 2000

_f32 = jnp.float32


def _mesh():
    return plsc.VectorSubcoreMesh(core_axis_name="c", subcore_axis_name="s")


_SC_PARAMS = pltpu.CompilerParams(use_tc_tiling_on_sc=False)


def _edge_lin_body(c_ref, w1_ref, b1_ref, w2_ref, b2_ref, c1_ref, c2_ref, had_ref):
    c = c_ref[...]
    c1 = jnp.dot(c, w1_ref[...], preferred_element_type=_f32) + b1_ref[...]
    c2 = jnp.dot(c, w2_ref[...], preferred_element_type=_f32) + b2_ref[...]
    c1_ref[...] = c1
    c2_ref[...] = c2
    had_ref[...] = c1 * c2


def _edge_lin(C, W1, b1, W2, b2):
    d = C.shape[1]
    grid = (_E // _BLK_E,)
    bs_c = pl.BlockSpec((_BLK_E, d), lambda i: (i, 0))
    bs_w = pl.BlockSpec((d, d), lambda i: (0, 0))
    bs_b = pl.BlockSpec((1, d), lambda i: (0, 0))
    return pl.pallas_call(
        _edge_lin_body,
        grid=grid,
        in_specs=[bs_c, bs_w, bs_b, bs_w, bs_b],
        out_specs=[bs_c, bs_c, bs_c],
        out_shape=[jax.ShapeDtypeStruct((_E, d), _f32)] * 3,
    )(C, W1, b1.reshape(1, d), W2, b2.reshape(1, d))


def _node_lin_body(x_ref, w3_ref, b3_ref, w4_ref, b4_ref, xw3_ref, xm_ref):
    xv = x_ref[...]
    xw3_ref[...] = jnp.dot(xv, w3_ref[...], preferred_element_type=_f32) + b3_ref[...]
    xm_ref[...] = jnp.dot(xv, w4_ref[...], preferred_element_type=_f32) + b4_ref[...]


def _node_lin(x, W3, b3, W4, b4):
    dn, d = W3.shape
    grid = (_N // _BLK_N,)
    bs_x = pl.BlockSpec((_BLK_N, dn), lambda i: (i, 0))
    bs_o = pl.BlockSpec((_BLK_N, d), lambda i: (i, 0))
    bs_w = pl.BlockSpec((dn, d), lambda i: (0, 0))
    bs_b = pl.BlockSpec((1, d), lambda i: (0, 0))
    return pl.pallas_call(
        _node_lin_body,
        grid=grid,
        in_specs=[bs_x, bs_w, bs_b, bs_w, bs_b],
        out_specs=[bs_o, bs_o],
        out_shape=[jax.ShapeDtypeStruct((_N, d), _f32)] * 2,
    )(x, W3, b3.reshape(1, d), W4, b4.reshape(1, d))


def _edge_out_body(c_ref, mm_ref, had_ref, dg_ref, id_ref, wa, wb, wc, wd, be,
                   out_ref, *, apply_gelu):
    acc = jnp.dot(c_ref[...], wa[...], preferred_element_type=_f32)
    acc += jnp.dot(mm_ref[...], wb[...], preferred_element_type=_f32)
    acc += jnp.dot(had_ref[...], wc[...], preferred_element_type=_f32)
    acc += jnp.dot(dg_ref[...] * id_ref[...], wd[...], preferred_element_type=_f32)
    acc += be[...]
    out_ref[...] = jax.nn.gelu(acc) if apply_gelu else acc


def _edge_out(C, mm, had, dg_raw, ident, We, be, apply_gelu):
    d = C.shape[1]
    do = We.shape[1]
    grid = (_E // _BLK_E,)
    bs_in = pl.BlockSpec((_BLK_E, d), lambda i: (i, 0))
    bs_id = pl.BlockSpec((_BLK_E, 1), lambda i: (i, 0))
    bs_w = pl.BlockSpec((d, do), lambda i: (0, 0))
    bs_b = pl.BlockSpec((1, do), lambda i: (0, 0))
    bs_o = pl.BlockSpec((_BLK_E, do), lambda i: (i, 0))
    return pl.pallas_call(
        functools.partial(_edge_out_body, apply_gelu=apply_gelu),
        grid=grid,
        in_specs=[bs_in, bs_in, bs_in, bs_in, bs_id, bs_w, bs_w, bs_w, bs_w, bs_b],
        out_specs=bs_o,
        out_shape=jax.ShapeDtypeStruct((_E, do), _f32),
    )(C, mm, had, dg_raw, ident, We[:d], We[d:2 * d], We[2 * d:3 * d], We[3 * d:], be.reshape(1, do))


def _node_out_body(x_ref, ag_ref, wa, wb, bn, out_ref, *, apply_gelu):
    acc = jnp.dot(x_ref[...], wa[...], preferred_element_type=_f32)
    agg = ag_ref[0] + ag_ref[1]
    acc += jnp.dot(agg, wb[...], preferred_element_type=_f32)
    acc += bn[...]
    out_ref[...] = jax.nn.gelu(acc) if apply_gelu else acc


def _node_out(x, agg2, Wn, bn, apply_gelu):
    dn = x.shape[1]
    d = agg2.shape[2]
    do = Wn.shape[1]
    grid = (_N // _BLK_N,)
    bs_x = pl.BlockSpec((_BLK_N, dn), lambda i: (i, 0))
    bs_a = pl.BlockSpec((2, _BLK_N, d), lambda i: (0, i, 0))
    bs_o = pl.BlockSpec((_BLK_N, do), lambda i: (i, 0))
    return pl.pallas_call(
        functools.partial(_node_out_body, apply_gelu=apply_gelu),
        grid=grid,
        in_specs=[bs_x,
                  bs_a,
                  pl.BlockSpec((dn, do), lambda i: (0, 0)),
                  pl.BlockSpec((d, do), lambda i: (0, 0)),
                  pl.BlockSpec((1, do), lambda i: (0, 0))],
        out_specs=bs_o,
        out_shape=jax.ShapeDtypeStruct((_N, do), _f32),
    )(x, agg2, Wn[:dn], Wn[dn:], bn.reshape(1, do))


def _seg_nodes_body(b_ref, x_ref, out_ref):
    i = pl.program_id(0)
    bid = lax.broadcasted_iota(jnp.int32, (1, _B), 1).astype(_f32)
    ind = (b_ref[...] == bid).astype(_f32)
    part = lax.dot_general(ind, x_ref[...], (((0,), (0,)), ((), ())),
                           preferred_element_type=_f32)

    @pl.when(i == 0)
    def _():
        out_ref[...] = jnp.zeros_like(out_ref)

    out_ref[...] += part


def _seg_nodes(batch_col, ox):
    dx = ox.shape[1]
    grid = (_N // _BLK_N,)
    return pl.pallas_call(
        _seg_nodes_body,
        grid=grid,
        in_specs=[pl.BlockSpec((_BLK_N, 1), lambda i: (i, 0)),
                  pl.BlockSpec((_BLK_N, dx), lambda i: (i, 0))],
        out_specs=pl.BlockSpec((_B, dx), lambda i: (0, 0)),
        out_shape=jax.ShapeDtypeStruct((_B, dx), _f32),
    )(batch_col, ox)


def _seg_edges_body(b_ref, id_ref, c_ref, hd_ref, ho_ref):
    i = pl.program_id(0)
    bid = lax.broadcasted_iota(jnp.int32, (1, _B), 1).astype(_f32)
    ind = (b_ref[...] == bid).astype(_f32)
    cv = c_ref[...]
    wc = cv * id_ref[...]
    s1 = lax.dot_general(ind, cv, (((0,), (0,)), ((), ())),
                         preferred_element_type=_f32)
    s2 = lax.dot_general(ind, wc, (((0,), (0,)), ((), ())),
                         preferred_element_type=_f32)

    @pl.when(i == 0)
    def _():
        hd_ref[...] = jnp.zeros_like(hd_ref)
        ho_ref[...] = jnp.zeros_like(ho_ref)

    hd_ref[...] += s2
    ho_ref[...] += s1 - s2


def _seg_edges(batch_col, ident, oc):
    dc = oc.shape[1]
    grid = (_E // _BLK_E,)
    return pl.pallas_call(
        _seg_edges_body,
        grid=grid,
        in_specs=[pl.BlockSpec((_BLK_E, 1), lambda i: (i, 0)),
                  pl.BlockSpec((_BLK_E, 1), lambda i: (i, 0)),
                  pl.BlockSpec((_BLK_E, dc), lambda i: (i, 0))],
        out_specs=[pl.BlockSpec((_B, dc), lambda i: (0, 0)),
                   pl.BlockSpec((_B, dc), lambda i: (0, 0))],
        out_shape=[jax.ShapeDtypeStruct((_B, dc), _f32)] * 2,
    )(batch_col, ident, oc)


def _mlp_body(hx_ref, hd_ref, ho_ref, w0a, w0b, w0c, b0, w1, b1, w2, b2, out_ref):
    h = jnp.dot(hx_ref[...], w0a[...], preferred_element_type=_f32)
    h += jnp.dot(hd_ref[...], w0b[...], preferred_element_type=_f32)
    h += jnp.dot(ho_ref[...], w0c[...], preferred_element_type=_f32)
    h = jax.nn.gelu(h + b0[...])
    h = jax.nn.gelu(jnp.dot(h, w1[...], preferred_element_type=_f32) + b1[...])
    out_ref[...] = jnp.dot(h, w2[...], preferred_element_type=_f32) + b2[...]


def _mlp(hx, hd, ho, params):
    dx = hx.shape[1]
    dc = hd.shape[1]
    w0 = params['fc0_W']
    outs = jax.ShapeDtypeStruct((_B, 1), _f32)
    return pl.pallas_call(
        _mlp_body,
        out_shape=outs,
    )(hx, hd, ho, w0[:dx], w0[dx:dx + dc], w0[dx + dc:],
      params['fc0_b'].reshape(1, -1),
      params['fc1_W'], params['fc1_b'].reshape(1, -1),
      params['fc2_W'], params['fc2_b'].reshape(1, -1))


_CH = 128
_T_CHUNKS = _T // _CH
_E_CHUNKS = _E // _CH


def _make_sc_prod(d):
    G1 = 4
    NGRP = _T_CHUNKS // G1
    nq = d // 16

    @functools.partial(
        pl.kernel,
        mesh=_mesh(),
        compiler_params=_SC_PARAMS,
        out_type=jax.ShapeDtypeStruct((_T, d), _f32),
        scratch_types=[pltpu.VMEM((G1, _CH), jnp.int32),
                       pltpu.VMEM((G1, _CH), jnp.int32),
                       pltpu.VMEM((G1 * _CH, d), _f32),
                       pltpu.VMEM((G1 * _CH, d), _f32),
                       pltpu.SemaphoreType.DMA],
    )
    def k(ti_hbm, tj_hbm, c1_hbm, c2_hbm, prod_hbm, ib, jb, c1b, c2b, sem):
        wid = lax.axis_index("s") * _NC + lax.axis_index("c")
        n_mine = NGRP // _NW + jnp.where(wid < (NGRP % _NW), 1, 0)

        @pl.loop(0, n_mine)
        def _(gi):
            goff = (gi * _NW + wid) * G1
            toff = goff * _CH
            pltpu.sync_copy(ti_hbm.at[pl.ds(goff, G1), :], ib)
            pltpu.sync_copy(tj_hbm.at[pl.ds(goff, G1), :], jb)
            descs = []
            for g in range(G1):
                descs.append(pltpu.async_copy(
                    c1_hbm.at[ib.at[g]], c1b.at[pl.ds(g * _CH, _CH), :], sem))
                descs.append(pltpu.async_copy(
                    c2_hbm.at[jb.at[g]], c2b.at[pl.ds(g * _CH, _CH), :], sem))
            for de in descs:
                de.wait()

            @pl.loop(0, G1 * _CH, unroll=2)
            def _(r):
                for q in range(nq):
                    sl = pl.ds(q * 16, 16)
                    c1b[r, sl] = c1b[r, sl] * c2b[r, sl]

            pltpu.sync_copy(c1b, prod_hbm.at[pl.ds(toff, G1 * _CH), :])

    return k


def _make_sc_scatter_mm(d):
    nb = d // 8
    nbc = nb // _NC
    G2 = 16
    NGRP = _T_CHUNKS // G2

    @functools.partial(
        pl.kernel,
        mesh=_mesh(),
        compiler_params=_SC_PARAMS,
        out_type=jax.ShapeDtypeStruct((_E, d), _f32),
        scratch_types=[pltpu.VMEM((G2, _CH), jnp.int32),
                       pltpu.VMEM((G2 * _CH, 8), _f32),
                       pltpu.VMEM_SHARED((_E, 8), _f32),
                       pltpu.SemaphoreType.DMA],
    )
    def k(tk_hbm, prod_hbm, zero_hbm, mm_hbm, kb, pb, acc, sem):
        core = lax.axis_index("c")
        sub = lax.axis_index("s")
        rows = _E // _NS
        n_mine = NGRP // _NS + jnp.where(sub < (NGRP % _NS), 1, 0)
        for g in range(nbc):
            bk = core * nbc + g

            @pl.when(sub == 0)
            def _():
                pltpu.sync_copy(zero_hbm, acc)

            plsc.subcore_barrier()

            @pl.loop(0, n_mine)
            def _(gi):
                goff = (gi * _NS + sub) * G2
                toff = goff * _CH
                pltpu.sync_copy(tk_hbm.at[pl.ds(goff, G2), :], kb)
                pltpu.sync_copy(prod_hbm.at[pl.ds(toff, G2 * _CH), pl.ds(bk * 8, 8)], pb)
                descs = []
                for g2 in range(G2):
                    descs.append(pltpu.async_copy(
                        pb.at[pl.ds(g2 * _CH, _CH), :], acc.at[kb.at[g2]], sem,
                        add=True))
                for de in descs:
                    de.wait()

            plsc.subcore_barrier()
            pltpu.sync_copy(acc.at[pl.ds(sub * rows, rows)],
                            mm_hbm.at[pl.ds(sub * rows, rows), pl.ds(bk * 8, 8)])
            plsc.subcore_barrier()

    return k


def _make_sc_edge(d):
    nrow = _N // _NS
    nq = d // 16
    G3 = 5
    NGRP = _E_CHUNKS // G3

    @functools.partial(
        pl.kernel,
        mesh=_mesh(),
        compiler_params=_SC_PARAMS,
        out_type=[jax.ShapeDtypeStruct((_E, d), _f32),
                  jax.ShapeDtypeStruct((_NC, _N, d), _f32)],
        scratch_types=[pltpu.VMEM((G3, _CH), jnp.int32),
                       pltpu.VMEM((G3, _CH), jnp.int32),
                       pltpu.VMEM((G3 * _CH, d), _f32),
                       pltpu.VMEM((G3 * _CH, d), _f32),
                       pltpu.VMEM_SHARED((_N, d), _f32),
                       pltpu.SemaphoreType.DMA],
    )
    def k(src_hbm, dst_hbm, xw3_hbm, xm_hbm, c_hbm, zero_hbm, dg_hbm, agg_hbm,
          sb, db, gb, cb, acc, sem):
        core = lax.axis_index("c")
        sub = lax.axis_index("s")
        wid = sub * _NC + core
        n_mine = NGRP // _NW + jnp.where(wid < (NGRP % _NW), 1, 0)

        @pl.when(sub == 0)
        def _():
            pltpu.sync_copy(zero_hbm, acc)

        plsc.subcore_barrier()

        @pl.loop(0, n_mine)
        def _(gi):
            goff = (gi * _NW + wid) * G3
            eoff = goff * _CH
            pltpu.sync_copy(src_hbm.at[pl.ds(goff, G3), :], sb)
            pltpu.sync_copy(dst_hbm.at[pl.ds(goff, G3), :], db)
            descs = [pltpu.async_copy(c_hbm.at[pl.ds(eoff, G3 * _CH), :], cb, sem)]
            for g in range(G3):
                descs.append(pltpu.async_copy(
                    xw3_hbm.at[sb.at[g]], gb.at[pl.ds(g * _CH, _CH), :], sem))
            for de in descs:
                de.wait()
            pltpu.sync_copy(gb, dg_hbm.at[pl.ds(eoff, G3 * _CH), :])
            descs = []
            for g in range(G3):
                descs.append(pltpu.async_copy(
                    xm_hbm.at[db.at[g]], gb.at[pl.ds(g * _CH, _CH), :], sem))
            for de in descs:
                de.wait()

            @pl.loop(0, G3 * _CH, unroll=2)
            def _(r):
                for q in range(nq):
                    sl = pl.ds(q * 16, 16)
                    cb[r, sl] = cb[r, sl] * gb[r, sl]

            descs = []
            for g in range(G3):
                descs.append(pltpu.async_copy(
                    cb.at[pl.ds(g * _CH, _CH), :], acc.at[sb.at[g]], sem,
                    add=True))
            for de in descs:
                de.wait()

        plsc.subcore_barrier()
        pltpu.sync_copy(acc.at[pl.ds(sub * nrow, nrow)],
                        agg_hbm.at[core, pl.ds(sub * nrow, nrow), :])

    return k


def kernel(x, edge_index3, edge_index2, edge_attr, batch, batch_edge,
           node_batch_edge, num_node, params):
    del node_batch_edge, num_node
    ti = edge_index3[0].astype(jnp.int32).reshape(_T // _CH, _CH)
    tj = edge_index3[1].astype(jnp.int32).reshape(_T // _CH, _CH)
    tk = edge_index3[2].astype(jnp.int32).reshape(_T // _CH, _CH)
    src = edge_index2[0].astype(jnp.int32).reshape(_E // _CH, _CH)
    dst = edge_index2[1].astype(jnp.int32).reshape(_E // _CH, _CH)
    ident = edge_attr[:, 0:1]
    zeros_e8 = jnp.zeros((_E, 8), _f32)

    sc_prod = {16: _make_sc_prod(16), 64: _make_sc_prod(64)}
    sc_mm = {16: _make_sc_scatter_mm(16), 64: _make_sc_scatter_mm(64)}
    sc_edge = {16: _make_sc_edge(16), 64: _make_sc_edge(64)}

    C = edge_attr
    xc = x
    out_x = [x]
    out_C = [C]
    for i in range(5):
        d = C.shape[1]
        apply_gelu = i < 4
        c1, c2, had = _edge_lin(C, params['l%d_W1' % i], params['l%d_b1' % i],
                                params['l%d_W2' % i], params['l%d_b2' % i])
        xw3, xm = _node_lin(xc, params['l%d_W3' % i], params['l%d_b3' % i],
                            params['l%d_W4' % i], params['l%d_b4' % i])
        prod = sc_prod[d](ti, tj, c1, c2)
        mm = sc_mm[d](tk, prod, zeros_e8)
        dg_raw, agg2 = sc_edge[d](src, dst, xw3, xm, C, jnp.zeros((_N, d), _f32))
        C = _edge_out(C, mm, had, dg_raw, ident,
                      params['l%d_We' % i], params['l%d_be' % i], apply_gelu)
        xc = _node_out(xc, agg2, params['l%d_Wn' % i], params['l%d_bn' % i],
                       apply_gelu)
        out_x.append(xc)
        out_C.append(C)

    ox = jnp.concatenate(out_x, axis=1)
    oc = jnp.concatenate(out_C, axis=1)
    batch_col = batch.astype(_f32).reshape(_N, 1)
    batch_e_col = batch_edge.astype(_f32).reshape(_E, 1)
    hx = _seg_nodes(batch_col, ox)
    hd, ho = _seg_edges(batch_e_col, ident, oc)
    return _mlp(hx, hd, ho, params)
